# baseline (speedup 1.0000x reference)
import functools

import jax
import jax.numpy as jnp
from jax import lax
from jax.experimental import pallas as pl
from jax.experimental.pallas import tpu as pltpu
from jax.experimental.pallas import tpu_sc as plsc

_NC = 2
_NS = 16
_NW = _NC * _NS

_B = 16384
_D = 32
_WIDE = 128
_RPW = _WIDE // _D
_VROWS = 1000000 * _D // _WIDE
_BPW = _B // _NW
_HB = _BPW // 2


def _gather_body(urow_hbm, irow_hbm, ut_hbm, it_hbm, out_u, out_i,
                 uidx_v, iidx_v, uw_v, iw_v, sem_u, sem_i):
    wid = lax.axis_index("s") * _NC + lax.axis_index("c")
    base = wid * _BPW
    pltpu.sync_copy(urow_hbm.at[pl.ds(base, _BPW)], uidx_v)
    pltpu.sync_copy(irow_hbm.at[pl.ds(base, _BPW)], iidx_v)
    for h in range(2):
        cu = pltpu.async_copy(ut_hbm.at[uidx_v.at[pl.ds(h * _HB, _HB)]],
                              uw_v, sem_u)
        ci = pltpu.async_copy(it_hbm.at[iidx_v.at[pl.ds(h * _HB, _HB)]],
                              iw_v, sem_i)
        cu.wait()
        ci.wait()
        pltpu.sync_copy(uw_v, out_u.at[pl.ds(base + h * _HB, _HB)])
        pltpu.sync_copy(iw_v, out_i.at[pl.ds(base + h * _HB, _HB)])


@functools.cache
def _make_gather():
    return pl.kernel(
        _gather_body,
        out_type=(
            jax.ShapeDtypeStruct((_B, _WIDE), jnp.float32),
            jax.ShapeDtypeStruct((_B, _WIDE), jnp.float32),
        ),
        mesh=plsc.VectorSubcoreMesh(core_axis_name="c", subcore_axis_name="s"),
        scratch_types=[
            pltpu.VMEM((_BPW,), jnp.int32),
            pltpu.VMEM((_BPW,), jnp.int32),
            pltpu.VMEM((_HB, _WIDE), jnp.float32),
            pltpu.VMEM((_HB, _WIDE), jnp.float32),
            pltpu.SemaphoreType.DMA,
            pltpu.SemaphoreType.DMA,
        ],
    )


def _select_subrow(wide, rem):
    acc = jnp.where(rem == 0, wide[:, 0:_D], 0.0)
    for r in range(1, _RPW):
        acc = acc + jnp.where(rem == r, wide[:, r * _D:(r + 1) * _D], 0.0)
    return acc


def _mlp_body(uw_ref, vw_ref, ru_ref, ri_ref, w1u_ref, w1i_ref, b1_ref,
              w2t_ref, b2_ref, w3_ref, b3_ref, out_ref):
    u = _select_subrow(uw_ref[:], ru_ref[:])
    v = _select_subrow(vw_ref[:], ri_ref[:])
    h = u @ w1u_ref[:] + v @ w1i_ref[:] + b1_ref[:]
    h = jnp.maximum(h, 0.0)
    h2 = jnp.maximum(h @ w2t_ref[:] + b2_ref[:], 0.0)
    out_ref[:] = jnp.sum(h2 * w3_ref[:], axis=1) + b3_ref[0]


def _mlp(uw, vw, ru, ri, w1u_t, w1i_t, b1, w2_t, b2, w3, b3, block_b=2048):
    nb = _B // block_b
    return pl.pallas_call(
        _mlp_body,
        grid=(nb,),
        in_specs=[
            pl.BlockSpec((block_b, _WIDE), lambda i: (i, 0)),
            pl.BlockSpec((block_b, _WIDE), lambda i: (i, 0)),
            pl.BlockSpec((block_b, 1), lambda i: (i, 0)),
            pl.BlockSpec((block_b, 1), lambda i: (i, 0)),
            pl.BlockSpec(w1u_t.shape, lambda i: (0, 0)),
            pl.BlockSpec(w1i_t.shape, lambda i: (0, 0)),
            pl.BlockSpec(b1.shape, lambda i: (0, 0)),
            pl.BlockSpec(w2_t.shape, lambda i: (0, 0)),
            pl.BlockSpec(b2.shape, lambda i: (0, 0)),
            pl.BlockSpec(w3.shape, lambda i: (0, 0)),
            pl.BlockSpec(b3.shape, lambda i: (0,)),
        ],
        out_specs=pl.BlockSpec((block_b,), lambda i: (i,)),
        out_shape=jax.ShapeDtypeStruct((_B,), jnp.float32),
    )(uw, vw, ru, ri, w1u_t, w1i_t, b1, w2_t, b2, w3, b3)


def kernel(user_ids, item_ids, user_table, item_table, W1, b1, W2, b2, W3, b3):
    ut_w = user_table.reshape(_VROWS, _WIDE)
    it_w = item_table.reshape(_VROWS, _WIDE)
    urow = lax.shift_right_logical(user_ids, 2)
    irow = lax.shift_right_logical(item_ids, 2)
    ru = (user_ids & 3)[:, None]
    ri = (item_ids & 3)[:, None]
    uw, vw = _make_gather()(urow, irow, ut_w, it_w)
    w1u_t = W1[:, :_D].T
    w1i_t = W1[:, _D:].T
    w2_t = W2.T
    return _mlp(uw, vw, ru, ri, w1u_t, w1i_t, b1[None, :], w2_t,
                b2[None, :], W3, b3)


# speedup vs baseline: 3.7117x; 3.7117x over previous
import functools

import jax
import jax.numpy as jnp
from jax import lax
from jax.experimental import pallas as pl
from jax.experimental.pallas import tpu as pltpu
from jax.experimental.pallas import tpu_sc as plsc

_NC = 2
_NS = 16
_NW = _NC * _NS

_B = 16384
_D = 32
_NROWS = 1000000
_BPW = _B // _NW
_G = 16
_NG = _BPW // _G
_LAST = (_NROWS // 128) * 128
_TAIL = _NROWS - _LAST


def _gather_body(uid_hbm, iid_hbm, ut_hbm, it_hbm, out_u, out_i,
                 uid_v, iid_v, slabs, obuf, sem):
    wid = lax.axis_index("s") * _NC + lax.axis_index("c")
    base = wid * _BPW
    pltpu.sync_copy(uid_hbm.at[pl.ds(base, _BPW)], uid_v)
    pltpu.sync_copy(iid_hbm.at[pl.ds(base, _BPW)], iid_v)

    jvec = lax.iota(jnp.int32, _G) * 128

    def one_table(tab_hbm, ids_v, out_hbm):
        def group(g, c):
            k0 = g * _G
            gvec = ids_v[pl.ds(k0, _G)]
            gcol = gvec & ~127
            for j in range(_G):
                col0 = pl.multiple_of(gcol[j], 128)
                pltpu.async_copy(
                    tab_hbm.at[:, pl.ds(col0, 128)],
                    slabs.at[:, pl.ds(j * 128, 128)], sem)

            pltpu.make_async_copy(
                tab_hbm.at[:, pl.ds(0, _G * 128)], slabs, sem).wait()

            colidx = jvec + (gvec & 127)
            for f in range(_D):
                rowv = jnp.full((_G,), f, jnp.int32)
                obuf[f, pl.ds(k0, _G)] = plsc.load_gather(slabs, [rowv, colidx])
            return c

        lax.fori_loop(0, _NG, group, 0)
        pltpu.sync_copy(obuf, out_hbm.at[:, pl.ds(base, _BPW)])

    one_table(ut_hbm, uid_v, out_u)
    one_table(it_hbm, iid_v, out_i)


@functools.cache
def _make_gather():
    return pl.kernel(
        _gather_body,
        out_type=(
            jax.ShapeDtypeStruct((_D, _B), jnp.float32),
            jax.ShapeDtypeStruct((_D, _B), jnp.float32),
        ),
        mesh=plsc.VectorSubcoreMesh(core_axis_name="c", subcore_axis_name="s"),
        scratch_types=[
            pltpu.VMEM((_BPW,), jnp.int32),
            pltpu.VMEM((_BPW,), jnp.int32),
            pltpu.VMEM((_D, _G * 128), jnp.float32),
            pltpu.VMEM((_D, _BPW), jnp.float32),
            pltpu.SemaphoreType.DMA,
        ],
        compiler_params=pltpu.CompilerParams(
            use_tc_tiling_on_sc=True, needs_layout_passes=False),
    )


def _mlp_body(ut_ref, vt_ref, w1u_ref, w1i_ref, b1_ref, w2_ref, b2_ref,
              w3_ref, b3_ref, out_ref):
    h = w1u_ref[:] @ ut_ref[:] + w1i_ref[:] @ vt_ref[:] + b1_ref[:]
    h = jnp.maximum(h, 0.0)
    h2 = jnp.maximum(w2_ref[:] @ h + b2_ref[:], 0.0)
    o = w3_ref[:] @ h2
    out_ref[:] = o[0] + b3_ref[0]


def _mlp(ut, vt, w1u, w1i, b1, w2, b2, w3, b3, block_b=2048):
    nb = _B // block_b
    return pl.pallas_call(
        _mlp_body,
        grid=(nb,),
        in_specs=[
            pl.BlockSpec((_D, block_b), lambda i: (0, i)),
            pl.BlockSpec((_D, block_b), lambda i: (0, i)),
            pl.BlockSpec(w1u.shape, lambda i: (0, 0)),
            pl.BlockSpec(w1i.shape, lambda i: (0, 0)),
            pl.BlockSpec(b1.shape, lambda i: (0, 0)),
            pl.BlockSpec(w2.shape, lambda i: (0, 0)),
            pl.BlockSpec(b2.shape, lambda i: (0, 0)),
            pl.BlockSpec(w3.shape, lambda i: (0, 0)),
            pl.BlockSpec(b3.shape, lambda i: (0,)),
        ],
        out_specs=pl.BlockSpec((block_b,), lambda i: (i,)),
        out_shape=jax.ShapeDtypeStruct((_B,), jnp.float32),
    )(ut, vt, w1u, w1i, b1, w2, b2, w3, b3)


def kernel(user_ids, item_ids, user_table, item_table, W1, b1, W2, b2, W3, b3):
    ut_t = user_table.T
    it_t = item_table.T
    u_t, v_t = _make_gather()(user_ids, item_ids, ut_t, it_t)
    w1u = W1[:, :_D]
    w1i = W1[:, _D:]
    return _mlp(u_t, v_t, w1u, w1i, b1[:, None], W2, b2[:, None], W3, b3)
